# Initial kernel scaffold; baseline (speedup 1.0000x reference)
import jax
import jax.numpy as jnp
from jax.experimental import pallas as pl


def kernel(lam, idx, wh_o):
    raise NotImplementedError("write your pallas kernel here")


# speedup vs baseline: 216.3881x; 216.3881x over previous
import functools

import jax
import jax.numpy as jnp
from jax import lax
from jax.experimental import pallas as pl
from jax.experimental.pallas import tpu as pltpu
from jax.experimental.pallas import tpu_sc as plsc

NC = 2
NS = 16
L = 16
NW = NC * NS

N_FEATS = 100000
NNZ = 3200000
N_OBS = 3200000

NBINS = 102400
SLICE = NBINS // NS
E = 2 * NNZ
NE = E // NW
CH = 10000
NO = N_OBS // NW
CG = 4000

_mesh = plsc.VectorSubcoreMesh(core_axis_name="c", subcore_axis_name="s")
_params = pltpu.CompilerParams(needs_layout_passes=False)


@functools.partial(
    pl.kernel, mesh=_mesh, compiler_params=_params,
    out_type=jax.ShapeDtypeStruct((NW, NBINS), jnp.float32),
    scratch_types=[
        pltpu.VMEM((NBINS,), jnp.float32),
        pltpu.VMEM((CH,), jnp.int32),
    ],
)
def _k1_histogram(idx_hbm, out_hbm, hist, idxbuf):
    c = lax.axis_index("c")
    s = lax.axis_index("s")
    wid = s * NC + c

    def zero_body(i, _):
        hist[pl.ds(i * L, L)] = jnp.zeros((L,), jnp.float32)
        return 0
    lax.fori_loop(0, NBINS // L, zero_body, 0)

    def chunk_body(k, _):
        base = wid * NE + k * CH
        pltpu.sync_copy(idx_hbm.at[pl.ds(base, CH)], idxbuf)

        def scat_body(i, _):
            iv = idxbuf[pl.ds(i * L, L)]
            plsc.addupdate_scatter(hist, [iv], jnp.ones((L,), jnp.float32))
            return 0
        lax.fori_loop(0, CH // L, scat_body, 0)
        return 0
    lax.fori_loop(0, NE // CH, chunk_body, 0)

    pltpu.sync_copy(hist, out_hbm.at[wid])


@functools.partial(
    pl.kernel, mesh=_mesh, compiler_params=_params,
    out_type=jax.ShapeDtypeStruct((N_OBS,), jnp.float32),
    scratch_types=[
        pltpu.VMEM((NBINS,), jnp.float32),
        pltpu.VMEM((CG,), jnp.int32),
        pltpu.VMEM((CG,), jnp.float32),
        pltpu.VMEM((SLICE,), jnp.float32),
        pltpu.VMEM((SLICE,), jnp.float32),
        pltpu.VMEM((NS * L,), jnp.float32),
        pltpu.VMEM_SHARED((NBINS,), jnp.float32),
        pltpu.VMEM_SHARED((NS * L,), jnp.float32),
        pltpu.SemaphoreType.DMA,
        pltpu.SemaphoreType.DMA,
    ],
)
def _k2_normalize_gather(parts_hbm, lam_hbm, wh_hbm, out_hbm,
                         ttab, whbuf, outbuf, tbuf, tbuf2, mbuf, st, smax,
                         sem0, sem1):
    c = lax.axis_index("c")
    s = lax.axis_index("s")
    wid = s * NC + c
    off = s * SLICE

    pltpu.sync_copy(parts_hbm.at[0, pl.ds(off, SLICE)], ttab.at[pl.ds(0, SLICE)])
    cp1 = pltpu.async_copy(parts_hbm.at[1, pl.ds(off, SLICE)], tbuf, sem0)
    cp2 = pltpu.async_copy(parts_hbm.at[2, pl.ds(off, SLICE)], tbuf2, sem1)
    for p in range(1, NW):
        use_a = (p % 2) == 1
        (cp1 if use_a else cp2).wait()
        src = tbuf if use_a else tbuf2

        def add_body(j, _):
            ttab[pl.ds(j * L, L)] = ttab[pl.ds(j * L, L)] + src[pl.ds(j * L, L)]
            return 0
        lax.fori_loop(0, SLICE // L, add_body, 0)
        if p + 2 < NW:
            if use_a:
                cp1 = pltpu.async_copy(parts_hbm.at[p + 2, pl.ds(off, SLICE)],
                                       tbuf, sem0)
            else:
                cp2 = pltpu.async_copy(parts_hbm.at[p + 2, pl.ds(off, SLICE)],
                                       tbuf2, sem1)

    pltpu.sync_copy(lam_hbm.at[pl.ds(off, SLICE)],
                    ttab.at[pl.ds(2 * SLICE, SLICE)])

    def max_body(j, m):
        return jnp.maximum(m, ttab[pl.ds(j * L, L)])
    mv = lax.fori_loop(0, SLICE // L, max_body, jnp.zeros((L,), jnp.float32))
    lmax = lax.reduce_max_p.bind(mv, axes=(0,))
    mbuf[pl.ds(0, L)] = jnp.full((L,), lmax)
    pltpu.sync_copy(mbuf.at[pl.ds(0, L)], smax.at[pl.ds(s * L, L)])
    plsc.subcore_barrier()
    pltpu.sync_copy(smax, mbuf)
    gv = mbuf[pl.ds(0, L)]
    for p in range(1, NS):
        gv = jnp.maximum(gv, mbuf[pl.ds(p * L, L)])
    inv = 1.0 / jnp.full((L,), lax.reduce_max_p.bind(gv, axes=(0,)))

    def t_body(j, _):
        d = ttab[pl.ds(j * L, L)]
        lm = ttab[pl.ds(2 * SLICE + j * L, L)]
        ttab[pl.ds(SLICE + j * L, L)] = (
            10.0 / (jnp.abs(lm * d * inv) + 1.0) + 1e-05)
        return 0
    lax.fori_loop(0, SLICE // L, t_body, 0)
    pltpu.sync_copy(ttab.at[pl.ds(SLICE, SLICE)], st.at[pl.ds(off, SLICE)])
    plsc.subcore_barrier()
    pltpu.sync_copy(st, ttab)

    def gchunk_body(k, _):
        base = wid * NO + k * CG
        pltpu.sync_copy(wh_hbm.at[pl.ds(base, CG)], whbuf)

        def g_body(i, _):
            iv = whbuf[pl.ds(i * L, L)]
            outbuf[pl.ds(i * L, L)] = plsc.load_gather(ttab, [iv])
            return 0
        lax.fori_loop(0, CG // L, g_body, 0)
        pltpu.sync_copy(outbuf, out_hbm.at[pl.ds(base, CG)])
        return 0
    lax.fori_loop(0, NO // CG, gchunk_body, 0)


def kernel(lam, idx, wh_o):
    idx_flat = jnp.reshape(idx, (E,))
    lam_pad = jnp.pad(lam, (0, NBINS - N_FEATS))
    parts = _k1_histogram(idx_flat)
    return _k2_normalize_gather(parts, lam_pad, wh_o)


# speedup vs baseline: 305.0306x; 1.4096x over previous
import functools

import jax
import jax.numpy as jnp
from jax import lax
from jax.experimental import pallas as pl
from jax.experimental.pallas import tpu as pltpu
from jax.experimental.pallas import tpu_sc as plsc

NC = 2
NS = 16
L = 16
NW = NC * NS

N_FEATS = 100000
NNZ = 3200000
N_OBS = 3200000

NBINS = 102400
SLICE = NBINS // NS
LAM_TAIL = N_FEATS - (NS - 1) * SLICE
E = 2 * NNZ
NE = E // NW
CH = 8000
NCH = NE // CH
US = 10
NO = N_OBS // NW
CG = 4000
NCG = NO // CG
UG = 10
UA = 8

_mesh = plsc.VectorSubcoreMesh(core_axis_name="c", subcore_axis_name="s")
_params = pltpu.CompilerParams(needs_layout_passes=False)


@functools.partial(
    pl.kernel, mesh=_mesh, compiler_params=_params,
    out_type=jax.ShapeDtypeStruct((NW, NBINS), jnp.float32),
    scratch_types=[
        pltpu.VMEM((NBINS,), jnp.float32),
        pltpu.VMEM((CH,), jnp.int32),
        pltpu.VMEM((CH,), jnp.int32),
        pltpu.SemaphoreType.DMA,
        pltpu.SemaphoreType.DMA,
    ],
)
def _k1_histogram(idx_hbm, out_hbm, hist, idxa, idxb, sem0, sem1):
    c = lax.axis_index("c")
    s = lax.axis_index("s")
    wid = s * NC + c

    def zero_body(i, _):
        for u in range(16):
            hist[pl.ds((i * 16 + u) * L, L)] = jnp.zeros((L,), jnp.float32)
        return 0
    lax.fori_loop(0, NBINS // (L * 16), zero_body, 0)

    bufs = (idxa, idxb)
    sems = (sem0, sem1)
    tile_base = wid * NE
    cps = [None, None]
    cps[0] = pltpu.async_copy(idx_hbm.at[pl.ds(tile_base, CH)], idxa, sem0)
    for k in range(NCH):
        cur = k % 2
        if k + 1 < NCH:
            nxt = (k + 1) % 2
            cps[nxt] = pltpu.async_copy(
                idx_hbm.at[pl.ds(tile_base + (k + 1) * CH, CH)],
                bufs[nxt], sems[nxt])
        cps[cur].wait()
        buf = bufs[cur]

        def scat_body(i, _):
            for u in range(US):
                iv = buf[pl.ds((i * US + u) * L, L)]
                plsc.addupdate_scatter(hist, [iv],
                                       jnp.ones((L,), jnp.float32))
            return 0
        lax.fori_loop(0, CH // (L * US), scat_body, 0)

    pltpu.sync_copy(hist, out_hbm.at[wid])


@functools.partial(
    pl.kernel, mesh=_mesh, compiler_params=_params,
    out_type=jax.ShapeDtypeStruct((N_OBS,), jnp.float32),
    scratch_types=[
        pltpu.VMEM((NBINS,), jnp.float32),
        pltpu.VMEM((CG,), jnp.int32),
        pltpu.VMEM((CG,), jnp.int32),
        pltpu.VMEM((SLICE,), jnp.float32),
        pltpu.VMEM((SLICE,), jnp.float32),
        pltpu.VMEM((NS * L,), jnp.float32),
        pltpu.VMEM_SHARED((NBINS,), jnp.float32),
        pltpu.VMEM_SHARED((NS * L,), jnp.float32),
        pltpu.SemaphoreType.DMA,
        pltpu.SemaphoreType.DMA,
        pltpu.SemaphoreType.DMA,
        pltpu.SemaphoreType.DMA,
    ],
)
def _k2_normalize_gather(parts_hbm, lam_hbm, wh_hbm, out_hbm,
                         ttab, wha, whb, tbuf, tbuf2, mbuf, st, smax,
                         sem0, sem1, sem2, sem3):
    c = lax.axis_index("c")
    s = lax.axis_index("s")
    wid = s * NC + c
    off = s * SLICE

    pltpu.sync_copy(parts_hbm.at[0, pl.ds(off, SLICE)], ttab.at[pl.ds(0, SLICE)])
    cp1 = pltpu.async_copy(parts_hbm.at[1, pl.ds(off, SLICE)], tbuf, sem0)
    cp2 = pltpu.async_copy(parts_hbm.at[2, pl.ds(off, SLICE)], tbuf2, sem1)
    for p in range(1, NW):
        use_a = (p % 2) == 1
        (cp1 if use_a else cp2).wait()
        src = tbuf if use_a else tbuf2

        def add_body(j, _):
            for u in range(UA):
                q = (j * UA + u) * L
                ttab[pl.ds(q, L)] = ttab[pl.ds(q, L)] + src[pl.ds(q, L)]
            return 0
        lax.fori_loop(0, SLICE // (L * UA), add_body, 0)
        if p + 2 < NW:
            if use_a:
                cp1 = pltpu.async_copy(parts_hbm.at[p + 2, pl.ds(off, SLICE)],
                                       tbuf, sem0)
            else:
                cp2 = pltpu.async_copy(parts_hbm.at[p + 2, pl.ds(off, SLICE)],
                                       tbuf2, sem1)

    @pl.when(s < NS - 1)
    def _():
        pltpu.sync_copy(lam_hbm.at[pl.ds(off, SLICE)],
                        ttab.at[pl.ds(2 * SLICE, SLICE)])

    @pl.when(s == NS - 1)
    def _():
        pltpu.sync_copy(lam_hbm.at[pl.ds(off, LAM_TAIL)],
                        ttab.at[pl.ds(2 * SLICE, LAM_TAIL)])

    def max_body(j, m):
        for u in range(UA):
            m = jnp.maximum(m, ttab[pl.ds((j * UA + u) * L, L)])
        return m
    mv = lax.fori_loop(0, SLICE // (L * UA), max_body,
                       jnp.zeros((L,), jnp.float32))
    lmax = lax.reduce_max_p.bind(mv, axes=(0,))
    mbuf[pl.ds(0, L)] = jnp.full((L,), lmax)
    pltpu.sync_copy(mbuf.at[pl.ds(0, L)], smax.at[pl.ds(s * L, L)])
    plsc.subcore_barrier()
    pltpu.sync_copy(smax, mbuf)
    gv = mbuf[pl.ds(0, L)]
    for p in range(1, NS):
        gv = jnp.maximum(gv, mbuf[pl.ds(p * L, L)])
    inv = 1.0 / jnp.full((L,), lax.reduce_max_p.bind(gv, axes=(0,)))

    def t_body(j, _):
        for u in range(UA):
            q = (j * UA + u) * L
            d = ttab[pl.ds(q, L)]
            lm = ttab[pl.ds(2 * SLICE + q, L)]
            ttab[pl.ds(SLICE + q, L)] = (
                10.0 / (jnp.abs(lm * d * inv) + 1.0) + 1e-05)
        return 0
    lax.fori_loop(0, SLICE // (L * UA), t_body, 0)
    pltpu.sync_copy(ttab.at[pl.ds(SLICE, SLICE)], st.at[pl.ds(off, SLICE)])
    plsc.subcore_barrier()
    pltpu.sync_copy(st, ttab)

    whs = (wha, whb)
    outs = (tbuf, tbuf2)
    isems = (sem0, sem1)
    osems = (sem2, sem3)
    obase = wid * NO
    cin = [None, None]
    cout = [None, None]
    cin[0] = pltpu.async_copy(wh_hbm.at[pl.ds(obase, CG)], wha, sem0)
    for k in range(NCG):
        cur = k % 2
        if k + 1 < NCG:
            nxt = (k + 1) % 2
            cin[nxt] = pltpu.async_copy(
                wh_hbm.at[pl.ds(obase + (k + 1) * CG, CG)],
                whs[nxt], isems[nxt])
        cin[cur].wait()
        if k >= 2:
            cout[cur].wait()
        wh = whs[cur]
        ob = outs[cur]

        def g_body(i, _):
            for u in range(UG):
                q = (i * UG + u) * L
                iv = wh[pl.ds(q, L)]
                ob[pl.ds(q, L)] = plsc.load_gather(ttab, [iv])
            return 0
        lax.fori_loop(0, CG // (L * UG), g_body, 0)
        cout[cur] = pltpu.async_copy(
            ob.at[pl.ds(0, CG)], out_hbm.at[pl.ds(obase + k * CG, CG)],
            osems[cur])
    cout[0].wait()
    cout[1].wait()


def kernel(lam, idx, wh_o):
    idx_flat = jnp.reshape(idx, (E,))
    parts = _k1_histogram(idx_flat)
    return _k2_normalize_gather(parts, lam, wh_o)


# speedup vs baseline: 507.0834x; 1.6624x over previous
import functools

import jax
import jax.numpy as jnp
from jax import lax
from jax.experimental import pallas as pl
from jax.experimental.pallas import tpu as pltpu
from jax.experimental.pallas import tpu_sc as plsc

NC = 2
NS = 16
L = 16
NW = NC * NS

N_FEATS = 100000
NNZ = 3200000
N_OBS = 3200000

NBINS = 102400
SLICE = NBINS // NS
LAM_TAIL = N_FEATS - (NS - 1) * SLICE
E = 2 * NNZ
NE = E // NW
CH = 8000
NCH = NE // CH
US = 10
NO = N_OBS // NW
CG = 4000
NCG = NO // CG
UG = 10
UA = 8

_mesh = plsc.VectorSubcoreMesh(core_axis_name="c", subcore_axis_name="s")
_params = pltpu.CompilerParams(needs_layout_passes=False,
                               use_tc_tiling_on_sc=False)


@functools.partial(
    pl.kernel, mesh=_mesh, compiler_params=_params,
    out_type=jax.ShapeDtypeStruct((NW, NBINS), jnp.float32),
    scratch_types=[
        pltpu.VMEM((NBINS,), jnp.float32),
        pltpu.VMEM((CH,), jnp.int32),
        pltpu.VMEM((CH,), jnp.int32),
        pltpu.SemaphoreType.DMA,
        pltpu.SemaphoreType.DMA,
    ],
)
def _k1_histogram(idx_hbm, out_hbm, hist, idxa, idxb, sem0, sem1):
    c = lax.axis_index("c")
    s = lax.axis_index("s")
    wid = s * NC + c

    @plsc.parallel_loop(0, NBINS // L, 1, unroll=16)
    def _(i):
        hist[pl.ds(i * L, L)] = jnp.zeros((L,), jnp.float32)

    bufs = (idxa, idxb)
    sems = (sem0, sem1)
    tile_base = s * (NNZ // NS)
    cps = [None, None]
    cps[0] = pltpu.async_copy(idx_hbm.at[c, pl.ds(tile_base, CH)], idxa, sem0)
    for k in range(NCH):
        cur = k % 2
        if k + 1 < NCH:
            nxt = (k + 1) % 2
            cps[nxt] = pltpu.async_copy(
                idx_hbm.at[c, pl.ds(tile_base + (k + 1) * CH, CH)],
                bufs[nxt], sems[nxt])
        cps[cur].wait()
        buf = bufs[cur]

        @plsc.parallel_loop(0, CH // L, 1, unroll=US)
        def _(i):
            iv = buf[pl.ds(i * L, L)]
            plsc.addupdate_scatter(hist, [iv], jnp.ones((L,), jnp.float32))

    pltpu.sync_copy(hist, out_hbm.at[wid])


@functools.partial(
    pl.kernel, mesh=_mesh, compiler_params=_params,
    out_type=jax.ShapeDtypeStruct((N_OBS,), jnp.float32),
    scratch_types=[
        pltpu.VMEM((NBINS,), jnp.float32),
        pltpu.VMEM((CG,), jnp.int32),
        pltpu.VMEM((CG,), jnp.int32),
        pltpu.VMEM((SLICE,), jnp.float32),
        pltpu.VMEM((SLICE,), jnp.float32),
        pltpu.VMEM((NS * L,), jnp.float32),
        pltpu.VMEM_SHARED((NBINS,), jnp.float32),
        pltpu.VMEM_SHARED((NS * L,), jnp.float32),
        pltpu.SemaphoreType.DMA,
        pltpu.SemaphoreType.DMA,
        pltpu.SemaphoreType.DMA,
        pltpu.SemaphoreType.DMA,
    ],
)
def _k2_normalize_gather(parts_hbm, lam_hbm, wh_hbm, out_hbm,
                         ttab, wha, whb, tbuf, tbuf2, mbuf, st, smax,
                         sem0, sem1, sem2, sem3):
    c = lax.axis_index("c")
    s = lax.axis_index("s")
    wid = s * NC + c
    off = s * SLICE

    pltpu.sync_copy(parts_hbm.at[0, pl.ds(off, SLICE)], ttab.at[pl.ds(0, SLICE)])
    cp1 = pltpu.async_copy(parts_hbm.at[1, pl.ds(off, SLICE)], tbuf, sem0)
    cp2 = pltpu.async_copy(parts_hbm.at[2, pl.ds(off, SLICE)], tbuf2, sem1)
    for p in range(1, NW):
        use_a = (p % 2) == 1
        (cp1 if use_a else cp2).wait()
        src = tbuf if use_a else tbuf2

        @plsc.parallel_loop(0, SLICE // L, 1, unroll=UA)
        def _(j):
            q = j * L
            ttab[pl.ds(q, L)] = ttab[pl.ds(q, L)] + src[pl.ds(q, L)]
        if p + 2 < NW:
            if use_a:
                cp1 = pltpu.async_copy(parts_hbm.at[p + 2, pl.ds(off, SLICE)],
                                       tbuf, sem0)
            else:
                cp2 = pltpu.async_copy(parts_hbm.at[p + 2, pl.ds(off, SLICE)],
                                       tbuf2, sem1)

    @pl.when(s < NS - 1)
    def _():
        pltpu.sync_copy(lam_hbm.at[pl.ds(off, SLICE)],
                        ttab.at[pl.ds(2 * SLICE, SLICE)])

    @pl.when(s == NS - 1)
    def _():
        pltpu.sync_copy(lam_hbm.at[pl.ds(off, LAM_TAIL)],
                        ttab.at[pl.ds(2 * SLICE, LAM_TAIL)])

    def max_body(j, m):
        return jnp.maximum(m, ttab[pl.ds(j * L, L)])
    mv = plsc.parallel_loop(0, SLICE // L, 1, unroll=UA,
                            carry=jnp.zeros((L,), jnp.float32))(max_body)
    lmax = lax.reduce_max_p.bind(mv, axes=(0,))
    mbuf[pl.ds(0, L)] = jnp.full((L,), lmax)
    pltpu.sync_copy(mbuf.at[pl.ds(0, L)], smax.at[pl.ds(s * L, L)])
    plsc.subcore_barrier()
    pltpu.sync_copy(smax, mbuf)
    gv = mbuf[pl.ds(0, L)]
    for p in range(1, NS):
        gv = jnp.maximum(gv, mbuf[pl.ds(p * L, L)])
    inv = 1.0 / jnp.full((L,), lax.reduce_max_p.bind(gv, axes=(0,)))

    @plsc.parallel_loop(0, SLICE // L, 1, unroll=UA)
    def _(j):
        q = j * L
        d = ttab[pl.ds(q, L)]
        lm = ttab[pl.ds(2 * SLICE + q, L)]
        ttab[pl.ds(SLICE + q, L)] = (
            10.0 / (jnp.abs(lm * d * inv) + 1.0) + 1e-05)
    pltpu.sync_copy(ttab.at[pl.ds(SLICE, SLICE)], st.at[pl.ds(off, SLICE)])
    plsc.subcore_barrier()
    pltpu.sync_copy(st, ttab)

    whs = (wha, whb)
    outs = (tbuf, tbuf2)
    isems = (sem0, sem1)
    osems = (sem2, sem3)
    obase = wid * NO
    cin = [None, None]
    cout = [None, None]
    cin[0] = pltpu.async_copy(wh_hbm.at[pl.ds(obase, CG)], wha, sem0)
    for k in range(NCG):
        cur = k % 2
        if k + 1 < NCG:
            nxt = (k + 1) % 2
            cin[nxt] = pltpu.async_copy(
                wh_hbm.at[pl.ds(obase + (k + 1) * CG, CG)],
                whs[nxt], isems[nxt])
        cin[cur].wait()
        if k >= 2:
            cout[cur].wait()
        wh = whs[cur]
        ob = outs[cur]

        @plsc.parallel_loop(0, CG // L, 1, unroll=UG)
        def _(i):
            q = i * L
            iv = wh[pl.ds(q, L)]
            ob[pl.ds(q, L)] = plsc.load_gather(ttab, [iv])
        cout[cur] = pltpu.async_copy(
            ob.at[pl.ds(0, CG)], out_hbm.at[pl.ds(obase + k * CG, CG)],
            osems[cur])
    cout[0].wait()
    cout[1].wait()


def kernel(lam, idx, wh_o):
    parts = _k1_histogram(idx)
    return _k2_normalize_gather(parts, lam, wh_o)


# speedup vs baseline: 600.6001x; 1.1844x over previous
import functools

import jax
import jax.numpy as jnp
from jax import lax
from jax.experimental import pallas as pl
from jax.experimental.pallas import tpu as pltpu
from jax.experimental.pallas import tpu_sc as plsc

NC = 2
NS = 16
L = 16
NW = NC * NS

N_FEATS = 100000
NNZ = 3200000
N_OBS = 3200000

NBINS = 102400
SLICE = NBINS // NS
LAM_TAIL = N_FEATS - (NS - 1) * SLICE
E = 2 * NNZ
NE = E // NW
CH = 8000
NCH = NE // CH
US = 10
NO = N_OBS // NW
CG = 4000
NCG = NO // CG
UG = 10
UA = 8

_mesh = plsc.VectorSubcoreMesh(core_axis_name="c", subcore_axis_name="s")
_params = pltpu.CompilerParams(needs_layout_passes=False,
                               use_tc_tiling_on_sc=False)


@functools.partial(
    pl.kernel, mesh=_mesh, compiler_params=_params,
    out_type=jax.ShapeDtypeStruct((NW, NBINS), jnp.float32),
    scratch_types=[
        pltpu.VMEM((NBINS,), jnp.float32),
        pltpu.VMEM((CH,), jnp.int32),
        pltpu.VMEM((CH,), jnp.int32),
        pltpu.SemaphoreType.DMA,
        pltpu.SemaphoreType.DMA,
    ],
)
def _k1_histogram(idx_hbm, out_hbm, hist, idxa, idxb, sem0, sem1):
    c = lax.axis_index("c")
    s = lax.axis_index("s")
    wid = s * NC + c

    @plsc.parallel_loop(0, NBINS // L, 1, unroll=16)
    def _(i):
        hist[pl.ds(i * L, L)] = jnp.zeros((L,), jnp.float32)

    bufs = (idxa, idxb)
    sems = (sem0, sem1)
    tile_base = wid * NE
    cps = [None, None]
    cps[0] = pltpu.async_copy(idx_hbm.at[pl.ds(tile_base, CH)], idxa, sem0)
    for k in range(NCH):
        cur = k % 2
        if k + 1 < NCH:
            nxt = (k + 1) % 2
            cps[nxt] = pltpu.async_copy(
                idx_hbm.at[pl.ds(tile_base + (k + 1) * CH, CH)],
                bufs[nxt], sems[nxt])
        cps[cur].wait()
        buf = bufs[cur]

        @plsc.parallel_loop(0, CH // L, 1, unroll=US)
        def _(i):
            iv = buf[pl.ds(i * L, L)]
            plsc.addupdate_scatter(hist, [iv], jnp.ones((L,), jnp.float32))

    pltpu.sync_copy(hist, out_hbm.at[wid])


@functools.partial(
    pl.kernel, mesh=_mesh, compiler_params=_params,
    out_type=jax.ShapeDtypeStruct((N_OBS,), jnp.float32),
    scratch_types=[
        pltpu.VMEM((NBINS,), jnp.float32),
        pltpu.VMEM((CG,), jnp.int32),
        pltpu.VMEM((CG,), jnp.int32),
        pltpu.VMEM((SLICE,), jnp.float32),
        pltpu.VMEM((SLICE,), jnp.float32),
        pltpu.VMEM((NS * L,), jnp.float32),
        pltpu.VMEM_SHARED((NBINS,), jnp.float32),
        pltpu.VMEM_SHARED((NS * L,), jnp.float32),
        pltpu.SemaphoreType.DMA,
        pltpu.SemaphoreType.DMA,
        pltpu.SemaphoreType.DMA,
        pltpu.SemaphoreType.DMA,
    ],
)
def _k2_normalize_gather(parts_hbm, lam_hbm, wh_hbm, out_hbm,
                         ttab, wha, whb, tbuf, tbuf2, mbuf, st, smax,
                         sem0, sem1, sem2, sem3):
    c = lax.axis_index("c")
    s = lax.axis_index("s")
    wid = s * NC + c
    off = s * SLICE

    pltpu.sync_copy(parts_hbm.at[0, pl.ds(off, SLICE)], ttab.at[pl.ds(0, SLICE)])
    cp1 = pltpu.async_copy(parts_hbm.at[1, pl.ds(off, SLICE)], tbuf, sem0)
    cp2 = pltpu.async_copy(parts_hbm.at[2, pl.ds(off, SLICE)], tbuf2, sem1)
    for p in range(1, NW):
        use_a = (p % 2) == 1
        (cp1 if use_a else cp2).wait()
        src = tbuf if use_a else tbuf2

        @plsc.parallel_loop(0, SLICE // L, 1, unroll=UA)
        def _(j):
            q = j * L
            ttab[pl.ds(q, L)] = ttab[pl.ds(q, L)] + src[pl.ds(q, L)]
        if p + 2 < NW:
            if use_a:
                cp1 = pltpu.async_copy(parts_hbm.at[p + 2, pl.ds(off, SLICE)],
                                       tbuf, sem0)
            else:
                cp2 = pltpu.async_copy(parts_hbm.at[p + 2, pl.ds(off, SLICE)],
                                       tbuf2, sem1)

    @pl.when(s < NS - 1)
    def _():
        pltpu.sync_copy(lam_hbm.at[pl.ds(off, SLICE)],
                        ttab.at[pl.ds(2 * SLICE, SLICE)])

    @pl.when(s == NS - 1)
    def _():
        pltpu.sync_copy(lam_hbm.at[pl.ds(off, LAM_TAIL)],
                        ttab.at[pl.ds(2 * SLICE, LAM_TAIL)])

    def max_body(j, m):
        return jnp.maximum(m, ttab[pl.ds(j * L, L)])
    mv = plsc.parallel_loop(0, SLICE // L, 1, unroll=UA,
                            carry=jnp.zeros((L,), jnp.float32))(max_body)
    lmax = lax.reduce_max_p.bind(mv, axes=(0,))
    mbuf[pl.ds(0, L)] = jnp.full((L,), lmax)
    pltpu.sync_copy(mbuf.at[pl.ds(0, L)], smax.at[pl.ds(s * L, L)])
    plsc.subcore_barrier()
    pltpu.sync_copy(smax, mbuf)
    gv = mbuf[pl.ds(0, L)]
    for p in range(1, NS):
        gv = jnp.maximum(gv, mbuf[pl.ds(p * L, L)])
    inv = 1.0 / jnp.full((L,), lax.reduce_max_p.bind(gv, axes=(0,)))

    @plsc.parallel_loop(0, SLICE // L, 1, unroll=UA)
    def _(j):
        q = j * L
        d = ttab[pl.ds(q, L)]
        lm = ttab[pl.ds(2 * SLICE + q, L)]
        ttab[pl.ds(SLICE + q, L)] = (
            10.0 / (jnp.abs(lm * d * inv) + 1.0) + 1e-05)
    pltpu.sync_copy(ttab.at[pl.ds(SLICE, SLICE)], st.at[pl.ds(off, SLICE)])
    plsc.subcore_barrier()
    pltpu.sync_copy(st, ttab)

    whs = (wha, whb)
    outs = (tbuf, tbuf2)
    isems = (sem0, sem1)
    osems = (sem2, sem3)
    obase = wid * NO
    cin = [None, None]
    cout = [None, None]
    cin[0] = pltpu.async_copy(wh_hbm.at[pl.ds(obase, CG)], wha, sem0)
    for k in range(NCG):
        cur = k % 2
        if k + 1 < NCG:
            nxt = (k + 1) % 2
            cin[nxt] = pltpu.async_copy(
                wh_hbm.at[pl.ds(obase + (k + 1) * CG, CG)],
                whs[nxt], isems[nxt])
        cin[cur].wait()
        if k >= 2:
            cout[cur].wait()
        wh = whs[cur]
        ob = outs[cur]

        @plsc.parallel_loop(0, CG // L, 1, unroll=UG)
        def _(i):
            q = i * L
            iv = wh[pl.ds(q, L)]
            ob[pl.ds(q, L)] = plsc.load_gather(ttab, [iv])
        cout[cur] = pltpu.async_copy(
            ob.at[pl.ds(0, CG)], out_hbm.at[pl.ds(obase + k * CG, CG)],
            osems[cur])
    cout[0].wait()
    cout[1].wait()


def kernel(lam, idx, wh_o):
    idx_perm = jnp.reshape(
        jnp.transpose(jnp.reshape(idx, (2, NNZ // 128, 128)), (1, 0, 2)),
        (E,))
    parts = _k1_histogram(idx_perm)
    return _k2_normalize_gather(parts, lam, wh_o)
